# baseline (speedup 1.0000x reference)
import functools

import jax
import jax.numpy as jnp
from jax import lax
from jax.experimental import pallas as pl
from jax.experimental.pallas import tpu as pltpu
from jax.experimental.pallas import tpu_sc as plsc

_NC = 2
_NS = 16
_NW = _NC * _NS


@functools.partial(jax.jit, static_argnums=(2, 3, 4))
def _gather(src_flat, table, B, D, C):
    b_per_w = B // _NW
    n_chunks = b_per_w // C
    assert n_chunks % 2 == 0 and n_chunks >= 4
    mesh = plsc.VectorSubcoreMesh(
        core_axis_name="c", subcore_axis_name="s",
        num_cores=_NC, num_subcores=_NS)

    @functools.partial(
        pl.kernel,
        out_type=jax.ShapeDtypeStruct((B, D), jnp.float32),
        mesh=mesh,
        scratch_types=[
            pltpu.VMEM((C,), jnp.int32),
            pltpu.VMEM((C,), jnp.int32),
            pltpu.VMEM((C, D), jnp.float32),
            pltpu.VMEM((C, D), jnp.float32),
            pltpu.SemaphoreType.DMA,
            pltpu.SemaphoreType.DMA,
            pltpu.SemaphoreType.DMA,
            pltpu.SemaphoreType.DMA,
            pltpu.SemaphoreType.DMA,
            pltpu.SemaphoreType.DMA,
        ],
        compiler_params=pltpu.CompilerParams(use_tc_tiling_on_sc=False),
    )
    def k(src_hbm, table_hbm, out_hbm,
          idx0, idx1, rows0, rows1,
          isem0, isem1, gsem0, gsem1, ssem0, ssem1):
        wid = lax.axis_index("s") * _NC + lax.axis_index("c")
        base = wid * b_per_w

        def step(i, idx_s, rows_s, isem_s, gsem_s, ssem_s,
                 idx_o, rows_o, isem_o, gsem_o, ssem_o):
            off = base + i * C
            pltpu.make_async_copy(table_hbm.at[idx_s], rows_s, gsem_s).wait()
            pltpu.async_copy(rows_s, out_hbm.at[pl.ds(off, C)], ssem_s)

            @pl.when(i + 2 < n_chunks)
            def _():
                pltpu.async_copy(
                    src_hbm.at[pl.ds(off + 2 * C, C)], idx_s, isem_s)

            @pl.when(i + 1 < n_chunks)
            def _():
                pltpu.make_async_copy(
                    src_hbm.at[pl.ds(off + C, C)], idx_o, isem_o).wait()

                @pl.when(i >= 1)
                def _():
                    pltpu.make_async_copy(
                        rows_o, out_hbm.at[pl.ds(off - C, C)], ssem_o).wait()

                pltpu.async_copy(table_hbm.at[idx_o], rows_o, gsem_o)

        pltpu.async_copy(src_hbm.at[pl.ds(base, C)], idx0, isem0)
        pltpu.async_copy(src_hbm.at[pl.ds(base + C, C)], idx1, isem1)
        pltpu.make_async_copy(src_hbm.at[pl.ds(base, C)], idx0, isem0).wait()
        pltpu.async_copy(table_hbm.at[idx0], rows0, gsem0)

        def pair(j, carry):
            i0 = 2 * j
            step(i0, idx0, rows0, isem0, gsem0, ssem0,
                 idx1, rows1, isem1, gsem1, ssem1)
            step(i0 + 1, idx1, rows1, isem1, gsem1, ssem1,
                 idx0, rows0, isem0, gsem0, ssem0)
            return carry

        lax.fori_loop(0, n_chunks // 2, pair, 0)

        pltpu.make_async_copy(
            rows0, out_hbm.at[pl.ds(base + (n_chunks - 2) * C, C)],
            ssem0).wait()
        pltpu.make_async_copy(
            rows1, out_hbm.at[pl.ds(base + (n_chunks - 1) * C, C)],
            ssem1).wait()

    return k(src_flat, table)


def kernel(src, table):
    D = table.shape[1]
    H = 4
    t = src.shape[1] // H
    outs = []
    for h in range(H):
        s = src[:, h * t:(h + 1) * t]
        Bh = src.shape[0] * t
        flat = s.reshape(Bh).astype(jnp.int32)
        outs.append(_gather(flat, table, Bh, D, 800).reshape(
            src.shape[0], t, D))
    return jnp.concatenate(outs, axis=1)


# speedup vs baseline: 2.0717x; 2.0717x over previous
import functools

import jax
import jax.numpy as jnp
from jax import lax
from jax.experimental import pallas as pl
from jax.experimental.pallas import tpu as pltpu
from jax.experimental.pallas import tpu_sc as plsc

_NC = 2
_NS = 16
_NW = _NC * _NS


@functools.partial(jax.jit, static_argnums=(2, 3, 4))
def _gather(src_flat, table, B, D, C):
    b_per_w = B // _NW
    n_chunks = b_per_w // C
    assert n_chunks % 2 == 0 and n_chunks >= 4
    mesh = plsc.VectorSubcoreMesh(
        core_axis_name="c", subcore_axis_name="s",
        num_cores=_NC, num_subcores=_NS)

    @functools.partial(
        pl.kernel,
        out_type=jax.ShapeDtypeStruct((B, D), jnp.float32),
        mesh=mesh,
        scratch_types=[
            pltpu.VMEM((C,), jnp.int32),
            pltpu.VMEM((C,), jnp.int32),
            pltpu.VMEM((C, D), jnp.float32),
            pltpu.VMEM((C, D), jnp.float32),
            pltpu.SemaphoreType.DMA,
            pltpu.SemaphoreType.DMA,
            pltpu.SemaphoreType.DMA,
            pltpu.SemaphoreType.DMA,
            pltpu.SemaphoreType.DMA,
            pltpu.SemaphoreType.DMA,
        ],
        compiler_params=pltpu.CompilerParams(use_tc_tiling_on_sc=False),
    )
    def k(src_hbm, table_hbm, out_hbm,
          idx0, idx1, rows0, rows1,
          isem0, isem1, gsem0, gsem1, ssem0, ssem1):
        wid = lax.axis_index("s") * _NC + lax.axis_index("c")
        base = wid * b_per_w

        def step(i, idx_s, rows_s, isem_s, gsem_s, ssem_s,
                 idx_o, rows_o, isem_o, gsem_o, ssem_o):
            off = base + i * C
            pltpu.make_async_copy(table_hbm.at[idx_s], rows_s, gsem_s).wait()
            pltpu.async_copy(rows_s, out_hbm.at[pl.ds(off, C)], ssem_s)

            @pl.when(i + 2 < n_chunks)
            def _():
                pltpu.async_copy(
                    src_hbm.at[pl.ds(off + 2 * C, C)], idx_s, isem_s)

            @pl.when(i + 1 < n_chunks)
            def _():
                pltpu.make_async_copy(
                    src_hbm.at[pl.ds(off + C, C)], idx_o, isem_o).wait()

                @pl.when(i >= 1)
                def _():
                    pltpu.make_async_copy(
                        rows_o, out_hbm.at[pl.ds(off - C, C)], ssem_o).wait()

                pltpu.async_copy(table_hbm.at[idx_o], rows_o, gsem_o)

        pltpu.async_copy(src_hbm.at[pl.ds(base, C)], idx0, isem0)
        pltpu.async_copy(src_hbm.at[pl.ds(base + C, C)], idx1, isem1)
        pltpu.make_async_copy(src_hbm.at[pl.ds(base, C)], idx0, isem0).wait()
        pltpu.async_copy(table_hbm.at[idx0], rows0, gsem0)

        def pair(j, carry):
            i0 = 2 * j
            step(i0, idx0, rows0, isem0, gsem0, ssem0,
                 idx1, rows1, isem1, gsem1, ssem1)
            step(i0 + 1, idx1, rows1, isem1, gsem1, ssem1,
                 idx0, rows0, isem0, gsem0, ssem0)
            return carry

        lax.fori_loop(0, n_chunks // 2, pair, 0)

        pltpu.make_async_copy(
            rows0, out_hbm.at[pl.ds(base + (n_chunks - 2) * C, C)],
            ssem0).wait()
        pltpu.make_async_copy(
            rows1, out_hbm.at[pl.ds(base + (n_chunks - 1) * C, C)],
            ssem1).wait()

    return k(src_flat, table)


def kernel(src, table):
    B = src.shape[0] * src.shape[1]
    D = table.shape[1]
    src_flat = src.reshape(B).astype(jnp.int32)
    out = _gather(src_flat, table, B, D, 800)
    return out.reshape(src.shape + (D,))


# speedup vs baseline: 2.5289x; 1.2207x over previous
import functools

import jax
import jax.numpy as jnp
from jax import lax
from jax.experimental import pallas as pl
from jax.experimental.pallas import tpu as pltpu
from jax.experimental.pallas import tpu_sc as plsc

_NC = 2
_NS = 16
_NW = _NC * _NS


@functools.partial(jax.jit, static_argnums=(2, 3))
def _gather(src_flat, pad, B, C):
    b_per_w = B // _NW
    n_chunks = b_per_w // C
    assert n_chunks % 2 == 0 and n_chunks >= 4
    mesh = plsc.VectorSubcoreMesh(
        core_axis_name="c", subcore_axis_name="s",
        num_cores=_NC, num_subcores=_NS)

    @functools.partial(
        pl.kernel,
        out_type=jax.ShapeDtypeStruct((B, 128), jnp.float32),
        mesh=mesh,
        scratch_types=[
            pltpu.VMEM((C,), jnp.int32),
            pltpu.VMEM((C,), jnp.int32),
            pltpu.VMEM((C, 128), jnp.float32),
            pltpu.VMEM((C, 128), jnp.float32),
            pltpu.SemaphoreType.DMA,
            pltpu.SemaphoreType.DMA,
            pltpu.SemaphoreType.DMA,
            pltpu.SemaphoreType.DMA,
            pltpu.SemaphoreType.DMA,
            pltpu.SemaphoreType.DMA,
        ],
        compiler_params=pltpu.CompilerParams(use_tc_tiling_on_sc=True),
    )
    def k(src_hbm, pad_hbm, out_hbm,
          idx0, idx1, rows0, rows1,
          isem0, isem1, gsem0, gsem1, ssem0, ssem1):
        wid = lax.axis_index("s") * _NC + lax.axis_index("c")
        base = wid * b_per_w

        def step(i, idx_s, rows_s, isem_s, gsem_s, ssem_s,
                 idx_o, rows_o, isem_o, gsem_o, ssem_o):
            off = base + i * C
            pltpu.make_async_copy(pad_hbm.at[idx_s], rows_s, gsem_s).wait()
            pltpu.async_copy(rows_s, out_hbm.at[pl.ds(off, C)], ssem_s)

            @pl.when(i + 2 < n_chunks)
            def _():
                pltpu.async_copy(
                    src_hbm.at[pl.ds(off + 2 * C, C)], idx_s, isem_s)

            @pl.when(i + 1 < n_chunks)
            def _():
                pltpu.make_async_copy(
                    src_hbm.at[pl.ds(off + C, C)], idx_o, isem_o).wait()

                @pl.when(i >= 1)
                def _():
                    pltpu.make_async_copy(
                        rows_o, out_hbm.at[pl.ds(off - C, C)], ssem_o).wait()

                pltpu.async_copy(pad_hbm.at[idx_o], rows_o, gsem_o)

        pltpu.async_copy(src_hbm.at[pl.ds(base, C)], idx0, isem0)
        pltpu.async_copy(src_hbm.at[pl.ds(base + C, C)], idx1, isem1)
        pltpu.make_async_copy(src_hbm.at[pl.ds(base, C)], idx0, isem0).wait()
        pltpu.async_copy(pad_hbm.at[idx0], rows0, gsem0)

        def pair(j, carry):
            i0 = 2 * j
            step(i0, idx0, rows0, isem0, gsem0, ssem0,
                 idx1, rows1, isem1, gsem1, ssem1)
            step(i0 + 1, idx1, rows1, isem1, gsem1, ssem1,
                 idx0, rows0, isem0, gsem0, ssem0)
            return carry

        lax.fori_loop(0, n_chunks // 2, pair, 0)

        pltpu.make_async_copy(
            rows0, out_hbm.at[pl.ds(base + (n_chunks - 2) * C, C)],
            ssem0).wait()
        pltpu.make_async_copy(
            rows1, out_hbm.at[pl.ds(base + (n_chunks - 1) * C, C)],
            ssem1).wait()

    return k(src_flat, pad)


def kernel(src, table):
    B = src.shape[0] * src.shape[1]
    D = table.shape[1]
    src_flat = src.reshape(B).astype(jnp.int32)
    pad = jnp.pad(table, ((0, 0), (0, 128 - D)))
    out = _gather(src_flat, pad, B, 256)
    return out[:, :D].reshape(src.shape + (D,))


# speedup vs baseline: 2.5321x; 1.0013x over previous
import functools

import jax
import jax.numpy as jnp
from jax import lax
from jax.experimental import pallas as pl
from jax.experimental.pallas import tpu as pltpu
from jax.experimental.pallas import tpu_sc as plsc

_NC = 2
_NS = 16
_NW = _NC * _NS


@functools.partial(jax.jit, static_argnums=(2, 3))
def _gather(src_flat, pad, B, C):
    b_per_w = B // _NW
    n_chunks = b_per_w // C
    assert n_chunks % 2 == 0 and n_chunks >= 4
    mesh = plsc.VectorSubcoreMesh(
        core_axis_name="c", subcore_axis_name="s",
        num_cores=_NC, num_subcores=_NS)

    @functools.partial(
        pl.kernel,
        out_type=jax.ShapeDtypeStruct((B, 128), jnp.float32),
        mesh=mesh,
        scratch_types=[
            pltpu.VMEM((C,), jnp.int32),
            pltpu.VMEM((C,), jnp.int32),
            pltpu.VMEM((C, 128), jnp.float32),
            pltpu.VMEM((C, 128), jnp.float32),
            pltpu.SemaphoreType.DMA,
            pltpu.SemaphoreType.DMA,
            pltpu.SemaphoreType.DMA,
            pltpu.SemaphoreType.DMA,
            pltpu.SemaphoreType.DMA,
            pltpu.SemaphoreType.DMA,
        ],
        compiler_params=pltpu.CompilerParams(use_tc_tiling_on_sc=True),
    )
    def k(src_hbm, pad_hbm, out_hbm,
          idx0, idx1, rows0, rows1,
          isem0, isem1, gsem0, gsem1, ssem0, ssem1):
        wid = lax.axis_index("s") * _NC + lax.axis_index("c")
        base = wid * b_per_w

        def step(i, idx_s, rows_s, isem_s, gsem_s, ssem_s,
                 idx_o, rows_o, isem_o, gsem_o, ssem_o):
            off = base + i * C
            pltpu.make_async_copy(pad_hbm.at[idx_s], rows_s, gsem_s).wait()
            pltpu.async_copy(rows_s, out_hbm.at[pl.ds(off, C)], ssem_s)

            @pl.when(i + 2 < n_chunks)
            def _():
                pltpu.async_copy(
                    src_hbm.at[pl.ds(off + 2 * C, C)], idx_s, isem_s)

            @pl.when(i + 1 < n_chunks)
            def _():
                pltpu.make_async_copy(
                    src_hbm.at[pl.ds(off + C, C)], idx_o, isem_o).wait()

                @pl.when(i >= 1)
                def _():
                    pltpu.make_async_copy(
                        rows_o, out_hbm.at[pl.ds(off - C, C)], ssem_o).wait()

                pltpu.async_copy(pad_hbm.at[idx_o], rows_o, gsem_o)

        pltpu.async_copy(src_hbm.at[pl.ds(base, C)], idx0, isem0)
        pltpu.async_copy(src_hbm.at[pl.ds(base + C, C)], idx1, isem1)
        pltpu.make_async_copy(src_hbm.at[pl.ds(base, C)], idx0, isem0).wait()
        pltpu.async_copy(pad_hbm.at[idx0], rows0, gsem0)

        def pair(j, carry):
            i0 = 2 * j
            step(i0, idx0, rows0, isem0, gsem0, ssem0,
                 idx1, rows1, isem1, gsem1, ssem1)
            step(i0 + 1, idx1, rows1, isem1, gsem1, ssem1,
                 idx0, rows0, isem0, gsem0, ssem0)
            return carry

        lax.fori_loop(0, n_chunks // 2, pair, 0)

        pltpu.make_async_copy(
            rows0, out_hbm.at[pl.ds(base + (n_chunks - 2) * C, C)],
            ssem0).wait()
        pltpu.make_async_copy(
            rows1, out_hbm.at[pl.ds(base + (n_chunks - 1) * C, C)],
            ssem1).wait()

    return k(src_flat, pad)


def kernel(src, table):
    B = src.shape[0] * src.shape[1]
    D = table.shape[1]
    src_flat = src.reshape(B).astype(jnp.int32)
    pad = jnp.pad(table, ((0, 0), (0, 128 - D)))
    out = _gather(src_flat, pad, B, 400)
    return out[:, :D].reshape(src.shape + (D,))
